# baseline (speedup 1.0000x reference)
import functools

import jax
import jax.numpy as jnp
from jax import lax
from jax.experimental import pallas as pl
from jax.experimental.pallas import tpu as pltpu
from jax.experimental.pallas import tpu_sc as plsc

VOCAB = 1000000
EMBED_DIM = 64
PADDED_DIM = 128
BATCH = 4096
HIST = 200
LANES = 16

NUM_CORES = 2
NUM_SUBCORES = 16
NW = NUM_CORES * NUM_SUBCORES

TOT = BATCH * HIST
PER_W = TOT // NW
CHUNK = 128
NCH = PER_W // CHUNK
NBUF = 4

NCOL = VOCAB // PADDED_DIM
VTAIL = VOCAB - NCOL * PADDED_DIM
BLOCKS_PER_W = -(-NCOL // NW)

_MESH = plsc.VectorSubcoreMesh(core_axis_name="c", subcore_axis_name="s")


def _worker_id():
    return lax.axis_index("s") * NUM_CORES + lax.axis_index("c")


@functools.partial(
    pl.kernel,
    out_type=jax.ShapeDtypeStruct((NCOL * EMBED_DIM, PADDED_DIM), jnp.float32),
    mesh=_MESH,
    scratch_types=[pltpu.SemaphoreType.DMA for _ in range(NBUF)],
)
def _sc_untile(tt_hbm, u_hbm, *sems):
    wid = _worker_id()

    def start(j, slot):
        s = wid + NW * j

        @pl.when(s < NCOL)
        def _():
            pltpu.async_copy(
                tt_hbm.at[:, pl.ds(s * PADDED_DIM, PADDED_DIM)],
                u_hbm.at[pl.ds(s * EMBED_DIM, EMBED_DIM)],
                sems[slot],
            )

    for j in range(NBUF):
        start(j, j)

    def body(g, _):
        for b in range(NBUF):
            j = g * NBUF + b
            s = wid + NW * j

            @pl.when(s < NCOL)
            def _():
                pltpu.make_async_copy(
                    tt_hbm.at[:, pl.ds(s * PADDED_DIM, PADDED_DIM)],
                    u_hbm.at[pl.ds(s * EMBED_DIM, EMBED_DIM)],
                    sems[b],
                ).wait()
                start(j + NBUF, b)

        return 0

    lax.fori_loop(0, -(-BLOCKS_PER_W // NBUF), body, 0)


@functools.partial(
    pl.kernel,
    out_type=jax.ShapeDtypeStruct((VOCAB, EMBED_DIM), jnp.float32),
    mesh=_MESH,
    compiler_params=pltpu.CompilerParams(
        use_tc_tiling_on_sc=False, needs_layout_passes=False
    ),
    scratch_types=[
        *[pltpu.VMEM((EMBED_DIM * PADDED_DIM,), jnp.float32) for _ in range(2)],
        *[pltpu.VMEM((PADDED_DIM, EMBED_DIM), jnp.float32) for _ in range(2)],
        pltpu.VMEM((PADDED_DIM, PADDED_DIM), jnp.float32),
        *[pltpu.SemaphoreType.DMA for _ in range(2)],
        *[pltpu.SemaphoreType.DMA for _ in range(2)],
        pltpu.SemaphoreType.DMA,
    ],
)
def _sc_repack(u_hbm, tail_hbm, tc_hbm, r0, r1, w0, w1, tl, rs0, rs1, ws0, ws1, ts):
    rbufs, wbufs, rsems, wsems = (r0, r1), (w0, w1), (rs0, rs1), (ws0, ws1)
    wid = _worker_id()

    lane = lax.iota(jnp.int32, LANES)
    bases = [(lane + (k * LANES)) * PADDED_DIM for k in range(EMBED_DIM // LANES)]

    def start_read(j, slot):
        s = wid + NW * j

        @pl.when(s < NCOL)
        def _():
            pltpu.async_copy(
                u_hbm.at[pl.ds(s * EMBED_DIM * PADDED_DIM, EMBED_DIM * PADDED_DIM)],
                rbufs[slot],
                rsems[slot],
            )

    for slot in range(2):
        start_read(slot, slot)

    def body(g, _):
        for sl in range(2):
            j = g * 2 + sl
            s = wid + NW * j

            @pl.when(s < NCOL)
            def _():
                @pl.when(j >= 2)
                def _():
                    pltpu.make_async_copy(
                        wbufs[sl], tc_hbm.at[pl.ds(0, PADDED_DIM)], wsems[sl]
                    ).wait()

                pltpu.make_async_copy(
                    u_hbm.at[
                        pl.ds(s * EMBED_DIM * PADDED_DIM, EMBED_DIM * PADDED_DIM)
                    ],
                    rbufs[sl],
                    rsems[sl],
                ).wait()

                def ubody(u, _):
                    for k in range(EMBED_DIM // LANES):
                        v = plsc.load_gather(rbufs[sl], [bases[k] + u])
                        wbufs[sl][u, pl.ds(k * LANES, LANES)] = v
                    return 0

                lax.fori_loop(0, PADDED_DIM, ubody, 0, unroll=8)
                start_read(j + 2, sl)
                pltpu.async_copy(
                    wbufs[sl],
                    tc_hbm.at[pl.ds(s * PADDED_DIM, PADDED_DIM)],
                    wsems[sl],
                )

        return 0

    lax.fori_loop(0, (BLOCKS_PER_W + 1) // 2, body, 0)

    for sl in range(2):
        pltpu.make_async_copy(
            wbufs[sl], tc_hbm.at[pl.ds(0, PADDED_DIM)], wsems[sl]
        ).wait()

    @pl.when(wid == NW - 1)
    def _():
        pltpu.sync_copy(tail_hbm, tl)
        pltpu.sync_copy(
            tl.at[pl.ds(0, VTAIL), pl.ds(0, EMBED_DIM)],
            tc_hbm.at[pl.ds(NCOL * PADDED_DIM, VTAIL)],
        )


@functools.partial(
    pl.kernel,
    out_type=jax.ShapeDtypeStruct((TOT, PADDED_DIM), jnp.float32),
    mesh=_MESH,
    compiler_params=pltpu.CompilerParams(use_tc_tiling_on_sc=False),
    scratch_types=[
        pltpu.VMEM((NCH, CHUNK), jnp.int32),
        *[pltpu.VMEM((CHUNK, EMBED_DIM), jnp.float32) for _ in range(NBUF)],
        *[pltpu.SemaphoreType.DMA for _ in range(NBUF)],
    ],
)
def _sc_gather(idx_hbm, table_hbm, out_hbm, idx_v, *bufs_and_sems):
    bufs = bufs_and_sems[:NBUF]
    sems = bufs_and_sems[NBUF:]

    wid = _worker_id()
    chunk0 = wid * NCH

    pltpu.sync_copy(idx_hbm.at[pl.ds(chunk0, NCH)], idx_v)

    for b in range(NBUF):
        pltpu.async_copy(table_hbm.at[idx_v.at[b]], bufs[b], sems[b])

    def body(g, _):
        for b in range(NBUF):
            j = g * NBUF + b
            pltpu.make_async_copy(
                table_hbm.at[idx_v.at[j]], bufs[b], sems[b]
            ).wait()
            pltpu.sync_copy(
                bufs[b],
                out_hbm.at[pl.ds((chunk0 + j) * CHUNK, CHUNK), pl.ds(0, EMBED_DIM)],
            )

            @pl.when(j + NBUF < NCH)
            def _():
                pltpu.async_copy(
                    table_hbm.at[idx_v.at[j + NBUF]], bufs[b], sems[b]
                )

        return 0

    lax.fori_loop(0, NCH // NBUF, body, 0)


@jax.jit
def kernel(input_indices, table):
    tt = table.T
    tail = jnp.pad(
        table[NCOL * PADDED_DIM :, :],
        ((0, PADDED_DIM - VTAIL), (0, PADDED_DIM - EMBED_DIM)),
    )
    u = _sc_untile(tt)
    tcomp = _sc_repack(u.reshape(NCOL * EMBED_DIM * PADDED_DIM), tail)
    idx = input_indices.reshape(TOT // CHUNK, CHUNK)
    out = _sc_gather(idx, tcomp)
    return out[:, :EMBED_DIM].reshape(BATCH, HIST, EMBED_DIM)


# speedup vs baseline: 5.1779x; 5.1779x over previous
import functools

import jax
import jax.numpy as jnp
from jax import lax
from jax.experimental import pallas as pl
from jax.experimental.pallas import tpu as pltpu
from jax.experimental.pallas import tpu_sc as plsc

VOCAB = 1000000
EMBED_DIM = 64
PADDED_DIM = 128
BATCH = 4096
HIST = 200
LANES = 16

NUM_CORES = 2
NUM_SUBCORES = 16
NW = NUM_CORES * NUM_SUBCORES

TOT = BATCH * HIST
PER_W = TOT // NW
CHUNK = 128
NCH = PER_W // CHUNK
NBUF = 4

NCOL = VOCAB // PADDED_DIM
VTAIL = VOCAB - NCOL * PADDED_DIM
BLOCKS_PER_W = -(-NCOL // NW)

_MESH = plsc.VectorSubcoreMesh(core_axis_name="c", subcore_axis_name="s")


def _worker_id():
    return lax.axis_index("s") * NUM_CORES + lax.axis_index("c")


@functools.partial(
    pl.kernel,
    out_type=jax.ShapeDtypeStruct((VOCAB // 2, PADDED_DIM), jnp.float32),
    mesh=_MESH,
    compiler_params=pltpu.CompilerParams(needs_layout_passes=False),
    scratch_types=[
        *[pltpu.VMEM((EMBED_DIM, PADDED_DIM), jnp.float32) for _ in range(2)],
        *[pltpu.VMEM((EMBED_DIM, PADDED_DIM), jnp.float32) for _ in range(2)],
        pltpu.VMEM((PADDED_DIM, PADDED_DIM), jnp.float32),
        *[pltpu.SemaphoreType.DMA for _ in range(2)],
        *[pltpu.SemaphoreType.DMA for _ in range(2)],
        pltpu.SemaphoreType.DMA,
    ],
)
def _sc_repack(tt_hbm, tail_hbm, tc_hbm, r0, r1, w0, w1, tl, rs0, rs1, ws0, ws1, ts):
    rbufs, wbufs, rsems, wsems = (r0, r1), (w0, w1), (rs0, rs1), (ws0, ws1)
    wid = _worker_id()

    lane = lax.iota(jnp.int32, LANES)
    zero = lane * 0
    rows = [lane + LANES * (k % 4) for k in range(PADDED_DIM // LANES)]

    def start_read(j, slot):
        s = wid + NW * j

        @pl.when(s < NCOL)
        def _():
            pltpu.async_copy(
                tt_hbm.at[:, pl.ds(s * PADDED_DIM, PADDED_DIM)],
                rbufs[slot],
                rsems[slot],
            )

    for slot in range(2):
        start_read(slot, slot)

    def body(g, _):
        for sl in range(2):
            j = g * 2 + sl
            s = wid + NW * j

            @pl.when(s < NCOL)
            def _():
                @pl.when(j >= 2)
                def _():
                    pltpu.make_async_copy(
                        wbufs[sl], tc_hbm.at[pl.ds(0, EMBED_DIM)], wsems[sl]
                    ).wait()

                pltpu.make_async_copy(
                    tt_hbm.at[:, pl.ds(s * PADDED_DIM, PADDED_DIM)],
                    rbufs[sl],
                    rsems[sl],
                ).wait()

                def qbody(q, _):
                    for k in range(PADDED_DIM // LANES):
                        v = plsc.load_gather(
                            rbufs[sl], [rows[k], zero + (2 * q + k // 4)]
                        )
                        wbufs[sl][q, pl.ds(k * LANES, LANES)] = v
                    return 0

                lax.fori_loop(0, EMBED_DIM, qbody, 0, unroll=8)
                start_read(j + 2, sl)
                pltpu.async_copy(
                    wbufs[sl],
                    tc_hbm.at[pl.ds(s * EMBED_DIM, EMBED_DIM)],
                    wsems[sl],
                )

        return 0

    lax.fori_loop(0, (BLOCKS_PER_W + 1) // 2, body, 0)

    for sl in range(2):
        pltpu.make_async_copy(
            wbufs[sl], tc_hbm.at[pl.ds(0, EMBED_DIM)], wsems[sl]
        ).wait()

    @pl.when(wid == NW - 1)
    def _():
        pltpu.sync_copy(tail_hbm, tl)
        for q in range(VTAIL // 2):
            for k in range(PADDED_DIM // LANES):
                v = plsc.load_gather(tl, [zero + (2 * q + k // 4), rows[k]])
                wbufs[0][q, pl.ds(k * LANES, LANES)] = v
        pltpu.sync_copy(
            wbufs[0].at[pl.ds(0, VTAIL // 2)],
            tc_hbm.at[pl.ds(NCOL * EMBED_DIM, VTAIL // 2)],
        )


@functools.partial(
    pl.kernel,
    out_type=jax.ShapeDtypeStruct((TOT, PADDED_DIM), jnp.float32),
    mesh=_MESH,
    compiler_params=pltpu.CompilerParams(use_tc_tiling_on_sc=False),
    scratch_types=[
        pltpu.VMEM((NCH, CHUNK), jnp.int32),
        *[pltpu.VMEM((CHUNK, EMBED_DIM), jnp.float32) for _ in range(NBUF)],
        *[pltpu.SemaphoreType.DMA for _ in range(NBUF)],
    ],
)
def _sc_gather(idx_hbm, table_hbm, out_hbm, idx_v, *bufs_and_sems):
    bufs = bufs_and_sems[:NBUF]
    sems = bufs_and_sems[NBUF:]

    wid = _worker_id()
    chunk0 = wid * NCH

    pltpu.sync_copy(idx_hbm.at[pl.ds(chunk0, NCH)], idx_v)

    for b in range(NBUF):
        pltpu.async_copy(table_hbm.at[idx_v.at[b]], bufs[b], sems[b])

    def body(g, _):
        for b in range(NBUF):
            j = g * NBUF + b
            pltpu.make_async_copy(
                table_hbm.at[idx_v.at[j]], bufs[b], sems[b]
            ).wait()
            pltpu.sync_copy(
                bufs[b],
                out_hbm.at[pl.ds((chunk0 + j) * CHUNK, CHUNK), pl.ds(0, EMBED_DIM)],
            )

            @pl.when(j + NBUF < NCH)
            def _():
                pltpu.async_copy(
                    table_hbm.at[idx_v.at[j + NBUF]], bufs[b], sems[b]
                )

        return 0

    lax.fori_loop(0, NCH // NBUF, body, 0)


@jax.jit
def kernel(input_indices, table):
    tt = table.T
    tail = jnp.pad(
        table[NCOL * PADDED_DIM :, :],
        ((0, PADDED_DIM - VTAIL), (0, PADDED_DIM - EMBED_DIM)),
    )
    tcomp = _sc_repack(tt, tail).reshape(VOCAB, EMBED_DIM)
    idx = input_indices.reshape(TOT // CHUNK, CHUNK)
    out = _sc_gather(idx, tcomp)
    return out[:, :EMBED_DIM].reshape(BATCH, HIST, EMBED_DIM)


# speedup vs baseline: 10.9628x; 2.1172x over previous
import functools

import jax
import jax.numpy as jnp
from jax import lax
from jax.experimental import pallas as pl
from jax.experimental.pallas import tpu as pltpu
from jax.experimental.pallas import tpu_sc as plsc

VOCAB = 1000000
EMBED_DIM = 64
PADDED_DIM = 128
BATCH = 4096
HIST = 200

NUM_CORES = 2
NUM_SUBCORES = 16
NW = NUM_CORES * NUM_SUBCORES

TOT = BATCH * HIST
PER_W = TOT // NW
CHUNK = 128
NCH = PER_W // CHUNK
NBUF = 4

_MESH = plsc.VectorSubcoreMesh(core_axis_name="c", subcore_axis_name="s")


def _worker_id():
    return lax.axis_index("s") * NUM_CORES + lax.axis_index("c")


@functools.partial(
    pl.kernel,
    out_type=jax.ShapeDtypeStruct((TOT, PADDED_DIM), jnp.float32),
    mesh=_MESH,
    compiler_params=pltpu.CompilerParams(use_tc_tiling_on_sc=False),
    scratch_types=[
        pltpu.VMEM((NCH, CHUNK), jnp.int32),
        *[pltpu.VMEM((CHUNK, EMBED_DIM), jnp.float32) for _ in range(NBUF)],
        *[pltpu.SemaphoreType.DMA for _ in range(NBUF)],
    ],
)
def _sc_gather(idx_hbm, table_hbm, out_hbm, idx_v, *bufs_and_sems):
    bufs = bufs_and_sems[:NBUF]
    sems = bufs_and_sems[NBUF:]

    wid = _worker_id()
    chunk0 = wid * NCH

    pltpu.sync_copy(idx_hbm.at[pl.ds(chunk0, NCH)], idx_v)

    for b in range(NBUF):
        pltpu.async_copy(table_hbm.at[idx_v.at[b]], bufs[b], sems[b])

    def body(g, _):
        for b in range(NBUF):
            j = g * NBUF + b
            pltpu.make_async_copy(
                table_hbm.at[idx_v.at[j]], bufs[b], sems[b]
            ).wait()
            pltpu.sync_copy(
                bufs[b],
                out_hbm.at[pl.ds((chunk0 + j) * CHUNK, CHUNK), pl.ds(0, EMBED_DIM)],
            )

            @pl.when(j + NBUF < NCH)
            def _():
                pltpu.async_copy(
                    table_hbm.at[idx_v.at[j + NBUF]], bufs[b], sems[b]
                )

        return 0

    lax.fori_loop(0, NCH // NBUF, body, 0)


@jax.jit
def kernel(input_indices, table):
    table2 = jnp.pad(table, ((0, 0), (0, PADDED_DIM - EMBED_DIM)))
    table2 = table2.reshape(2 * VOCAB, EMBED_DIM)
    idx = (input_indices * 2).reshape(TOT // CHUNK, CHUNK)
    out = _sc_gather(idx, table2)
    return out[:, :EMBED_DIM].reshape(BATCH, HIST, EMBED_DIM)


# speedup vs baseline: 10.9676x; 1.0004x over previous
import functools

import jax
import jax.numpy as jnp
from jax import lax
from jax.experimental import pallas as pl
from jax.experimental.pallas import tpu as pltpu
from jax.experimental.pallas import tpu_sc as plsc

VOCAB = 1000000
EMBED_DIM = 64
PADDED_DIM = 128
BATCH = 4096
HIST = 200

NUM_CORES = 2
NUM_SUBCORES = 16
NW = NUM_CORES * NUM_SUBCORES

TOT = BATCH * HIST
PER_W = TOT // NW
CHUNK = 256
NCH = PER_W // CHUNK
NBUF = 4

_MESH = plsc.VectorSubcoreMesh(core_axis_name="c", subcore_axis_name="s")


def _worker_id():
    return lax.axis_index("s") * NUM_CORES + lax.axis_index("c")


@functools.partial(
    pl.kernel,
    out_type=jax.ShapeDtypeStruct((TOT, PADDED_DIM), jnp.float32),
    mesh=_MESH,
    compiler_params=pltpu.CompilerParams(use_tc_tiling_on_sc=False),
    scratch_types=[
        pltpu.VMEM((NCH, CHUNK), jnp.int32),
        *[pltpu.VMEM((CHUNK, EMBED_DIM), jnp.float32) for _ in range(NBUF)],
        *[pltpu.SemaphoreType.DMA for _ in range(NBUF)],
    ],
)
def _sc_gather(idx_hbm, table_hbm, out_hbm, idx_v, *bufs_and_sems):
    bufs = bufs_and_sems[:NBUF]
    sems = bufs_and_sems[NBUF:]

    wid = _worker_id()
    chunk0 = wid * NCH

    pltpu.sync_copy(idx_hbm.at[pl.ds(chunk0, NCH)], idx_v)

    for b in range(NBUF):
        pltpu.async_copy(table_hbm.at[idx_v.at[b]], bufs[b], sems[b])

    def body(g, _):
        for b in range(NBUF):
            j = g * NBUF + b
            pltpu.make_async_copy(
                table_hbm.at[idx_v.at[j]], bufs[b], sems[b]
            ).wait()
            pltpu.sync_copy(
                bufs[b],
                out_hbm.at[pl.ds((chunk0 + j) * CHUNK, CHUNK), pl.ds(0, EMBED_DIM)],
            )

            @pl.when(j + NBUF < NCH)
            def _():
                pltpu.async_copy(
                    table_hbm.at[idx_v.at[j + NBUF]], bufs[b], sems[b]
                )

        return 0

    lax.fori_loop(0, NCH // NBUF, body, 0)


@jax.jit
def kernel(input_indices, table):
    table2 = jnp.pad(table, ((0, 0), (0, PADDED_DIM - EMBED_DIM)))
    table2 = table2.reshape(2 * VOCAB, EMBED_DIM)
    idx = (input_indices * 2).reshape(TOT // CHUNK, CHUNK)
    out = _sc_gather(idx, table2)
    return out[:, :EMBED_DIM].reshape(BATCH, HIST, EMBED_DIM)
